# Initial kernel scaffold; baseline (speedup 1.0000x reference)
import jax
import jax.numpy as jnp
from jax.experimental import pallas as pl


def kernel(feats, points3d, conf_logits, W1, b1, g1, be1, W2, b2, Wp1, bp1, Wp2, bp2, Wf, bf, gf, bef):
    raise NotImplementedError("write your pallas kernel here")


# speedup vs baseline: 1.4680x; 1.4680x over previous
import functools

import jax
import jax.numpy as jnp
from jax import lax
from jax.experimental import pallas as pl
from jax.experimental.pallas import tpu as pltpu
from jax.experimental.pallas import tpu_sc as plsc

G = 32
M = G * G * G
VS = 1.0 / G
N = 65536
D = 1024
P = 128


BN = 256
NB = N // BN


def _mlp_body(feats_ref, pts_ref, conf_ref, W1_ref, b1_ref, g1_ref, be1_ref,
              W2_ref, b2_ref, hw_ref, w_ref, vid_ref):
    x = feats_ref[...]
    h = jnp.dot(x, W1_ref[...], preferred_element_type=jnp.float32) + b1_ref[...]
    mu = jnp.mean(h, axis=-1, keepdims=True)
    var = jnp.mean((h - mu) ** 2, axis=-1, keepdims=True)
    h = (h - mu) / jnp.sqrt(var + 1e-5) * g1_ref[...] + be1_ref[...]
    h = jax.nn.gelu(h)
    h = jnp.dot(h, W2_ref[...], preferred_element_type=jnp.float32) + b2_ref[...]
    w = 1.0 + jnp.exp(conf_ref[...])
    hw_ref[...] = h * w
    w_ref[...] = w
    pts = pts_ref[...]
    ij = jnp.clip(jnp.floor(pts / VS), 0, G - 1).astype(jnp.int32)
    vid_ref[...] = ij[:, 0:1] * (G * G) + ij[:, 1:2] * G + ij[:, 2:3]


def _mlp_call(feats, pts, conf2, W1, b1, g1, be1, W2, b2):
    return pl.pallas_call(
        _mlp_body,
        grid=(NB,),
        in_specs=[
            pl.BlockSpec((BN, D), lambda i: (i, 0)),
            pl.BlockSpec((BN, 3), lambda i: (i, 0)),
            pl.BlockSpec((BN, 1), lambda i: (i, 0)),
            pl.BlockSpec((D, D), lambda i: (0, 0)),
            pl.BlockSpec((1, D), lambda i: (0, 0)),
            pl.BlockSpec((1, D), lambda i: (0, 0)),
            pl.BlockSpec((1, D), lambda i: (0, 0)),
            pl.BlockSpec((D, D), lambda i: (0, 0)),
            pl.BlockSpec((1, D), lambda i: (0, 0)),
        ],
        out_specs=[
            pl.BlockSpec((BN, D), lambda i: (i, 0)),
            pl.BlockSpec((BN, 1), lambda i: (i, 0)),
            pl.BlockSpec((BN, 1), lambda i: (i, 0)),
        ],
        out_shape=[
            jax.ShapeDtypeStruct((N, D), jnp.float32),
            jax.ShapeDtypeStruct((N, 1), jnp.float32),
            jax.ShapeDtypeStruct((N, 1), jnp.int32),
        ],
    )(feats, pts, conf2, W1, b1, g1, be1, W2, b2)


NSC = 2
NT = 16
HALF = M // NSC
PPTS = N // NT
BK = 64
NBK = HALF // BK
ROUNDS = NBK // NT
CAPB = 48
LISTW = NBK * CAPB
CTOT = NT * CAPB


def _take16(x, idx):
    dn = lax.GatherDimensionNumbers(offset_dims=(), collapsed_slice_dims=(0,),
                                    start_index_map=(0,))
    return lax.gather(x, idx[:, None], dn, (1,),
                      mode=lax.GatherScatterMode.PROMISE_IN_BOUNDS)


def _sc_body(hw_hbm, w_hbm, vid_hbm, agg_hbm, den_hbm,
             lists_sh, cnts_sh, vid_l, listb, cnts, cbuf, pstage, cpacked,
             pbufg, acc, rowbuf, wbuf, den64):
    c = lax.axis_index("c")
    s = lax.axis_index("s")
    half0 = c * HALF
    zero16f = jnp.zeros((16,), jnp.float32)
    zero16i = jnp.zeros((16,), jnp.int32)
    iota16 = lax.iota(jnp.int32, 16)

    pltpu.sync_copy(vid_hbm.at[pl.ds(s * PPTS, PPTS)], vid_l)

    def zc(i, _):
        cnts[pl.ds(i * 16, 16)] = zero16i
        return 0
    lax.fori_loop(0, (NBK + 16) // 16, zc, 0)

    def prod(i, _):
        v = vid_l[pl.ds(i * 16, 16)]
        rel = v - half0
        valid = (rel >= 0) & (rel < HALF)
        b = jnp.where(valid, rel >> 6, NBK)
        packed = ((v & (BK - 1)) << 16) | (s * PPTS + i * 16 + iota16)
        sb, sp = plsc.sort_key_val(b, packed)
        prev = _take16(sb, jnp.maximum(iota16 - 1, 0))
        newrun = (sb != prev) | (iota16 == 0)
        runpos = plsc.cummax(jnp.where(newrun, iota16, 0))
        rank = iota16 - runpos
        nxt = _take16(sb, jnp.minimum(iota16 + 1, 15))
        runend = (sb != nxt) | (iota16 == 15)
        cb = plsc.load_gather(cnts, [sb])
        pos = sb * CAPB + jnp.minimum(cb + rank, CAPB - 1)
        plsc.store_scatter(listb, [pos], sp)
        plsc.store_scatter(cnts, [sb], cb + rank + 1, mask=runend)
        return 0
    lax.fori_loop(0, PPTS // 16, prod, 0)

    pltpu.sync_copy(listb.at[pl.ds(0, LISTW)], lists_sh.at[pl.ds(s * LISTW, LISTW)])
    pltpu.sync_copy(cnts.at[pl.ds(0, NBK)], cnts_sh.at[pl.ds(s * NBK, NBK)])
    plsc.subcore_barrier()
    pltpu.sync_copy(cnts_sh, cbuf)

    def rnd(r, _):
        b = r * NT + s
        v0 = half0 + b * BK

        def za(i, _):
            acc[i // (D // 16), pl.ds((i % (D // 16)) * 16, 16)] = zero16f
            return 0
        lax.fori_loop(0, BK * D // 16, za, 0)
        for i in range(BK // 16):
            den64[pl.ds(i * 16, 16)] = zero16f

        for s2 in range(NT):
            pltpu.sync_copy(lists_sh.at[pl.ds(s2 * LISTW + b * CAPB, CAPB)],
                            pstage.at[pl.ds(s2 * CAPB, CAPB)])
        cvec = plsc.load_gather(cbuf, [iota16 * NBK + b])

        cnt = zero16i
        for k in range(CTOT // 16):
            t = k % (CAPB // 16)
            c2 = jnp.minimum(cvec[k // (CAPB // 16)], CAPB)
            vec = pstage[pl.ds(k * 16, 16)]
            m = (t * 16 + iota16) < c2
            pos = cnt + plsc.cumsum(m.astype(jnp.int32)) - 1
            plsc.store_scatter(cpacked, [pos], vec, mask=m)
            cnt = cnt + plsc.all_reduce_population_count(m)
        ctot = jnp.max(cnt)

        def bg(k, _):
            pbufg[pl.ds(k * 16, 16)] = cpacked[pl.ds(k * 16, 16)] & 0xFFFF
            return 0
        lax.fori_loop(0, CTOT // 16, bg, 0)

        nc = (ctot + 15) // 16

        def chunk(j, _):
            pltpu.sync_copy(hw_hbm.at[pbufg.at[pl.ds(j * 16, 16)]], rowbuf)
            pltpu.sync_copy(w_hbm.at[pbufg.at[pl.ds(j * 16, 16)]], wbuf)
            pkv = cpacked[pl.ds(j * 16, 16)]
            wv = wbuf[...]
            kmax = jnp.minimum(ctot - j * 16, 16)

            key = jnp.where(iota16 < kmax, pkv >> 16, BK)
            sk, sw = plsc.sort_key_val(key, wv)
            prev = _take16(sk, jnp.maximum(iota16 - 1, 0))
            newrun = (sk != prev) | (iota16 == 0)
            runpos = plsc.cummax(jnp.where(newrun, iota16, 0))
            nxt = _take16(sk, jnp.minimum(iota16 + 1, 15))
            runend = (sk != nxt) | (iota16 == 15)
            cs = plsc.cumsum(sw)
            prevcs = _take16(cs, jnp.maximum(runpos - 1, 0))
            seg = cs - jnp.where(runpos == 0, 0.0, prevcs)
            plsc.addupdate_scatter(den64, [sk], seg,
                                   mask=runend & (sk < BK))

            for k in range(16):
                @pl.when(j * 16 + k < ctot)
                def _add():
                    lv = pkv[k] >> 16
                    for q in range(D // 16):
                        plsc.addupdate(acc.at[lv, pl.ds(q * 16, 16)],
                                       rowbuf[k, pl.ds(q * 16, 16)])
            return 0
        lax.fori_loop(0, nc, chunk, 0)

        pltpu.sync_copy(acc, agg_hbm.at[pl.ds(v0, BK)])
        pltpu.sync_copy(den64, den_hbm.at[pl.ds(v0, BK)])
        return 0
    lax.fori_loop(0, ROUNDS, rnd, 0)


def _sc_call(hw, w, vid):
    mesh = plsc.VectorSubcoreMesh(core_axis_name="c", subcore_axis_name="s")
    f = pl.kernel(
        _sc_body,
        out_type=(
            jax.ShapeDtypeStruct((M, D), jnp.float32),
            jax.ShapeDtypeStruct((M,), jnp.float32),
        ),
        mesh=mesh,
        compiler_params=pltpu.CompilerParams(needs_layout_passes=False),
        scratch_types=[
            pltpu.VMEM_SHARED((NT * LISTW,), jnp.int32),
            pltpu.VMEM_SHARED((NT * NBK,), jnp.int32),
            pltpu.VMEM((PPTS,), jnp.int32),
            pltpu.VMEM((LISTW + CAPB,), jnp.int32),
            pltpu.VMEM((NBK + 16,), jnp.int32),
            pltpu.VMEM((NT * NBK,), jnp.int32),
            pltpu.VMEM((CTOT,), jnp.int32),
            pltpu.VMEM((CTOT,), jnp.int32),
            pltpu.VMEM((CTOT,), jnp.int32),
            pltpu.VMEM((BK, D), jnp.float32),
            pltpu.VMEM((16, D), jnp.float32),
            pltpu.VMEM((16,), jnp.float32),
            pltpu.VMEM((BK,), jnp.float32),
        ],
    )
    return f(hw, w, vid)


BM = 512
MB = M // BM


def _fuse_body(agg_ref, denp_ref, Wp1_ref, bp1_ref, Wp2_ref, bp2_ref,
               Wf_ref, bf_ref, gf_ref, bef_ref, out_ref):
    i = pl.program_id(0)
    den = denp_ref[0, 0]
    agg = agg_ref[...] / (den[:, None] + 1e-8)
    gi = i * BM + lax.broadcasted_iota(jnp.int32, (BM, 1), 0)
    c0 = ((gi // (G * G)).astype(jnp.float32) + 0.5) * VS
    c1 = (((gi // G) % G).astype(jnp.float32) + 0.5) * VS
    c2 = ((gi % G).astype(jnp.float32) + 0.5) * VS
    Wp1 = Wp1_ref[...]
    pe = c0 * Wp1[0:1, :] + c1 * Wp1[1:2, :] + c2 * Wp1[2:3, :] + bp1_ref[...]
    pe = jax.nn.silu(pe)
    pe = jnp.dot(pe, Wp2_ref[...], preferred_element_type=jnp.float32) + bp2_ref[...]
    Wf = Wf_ref[...]
    tok = (jnp.dot(agg, Wf[:D, :], preferred_element_type=jnp.float32)
           + jnp.dot(pe, Wf[D:, :], preferred_element_type=jnp.float32)
           + bf_ref[...])
    mu = jnp.mean(tok, axis=-1, keepdims=True)
    var = jnp.mean((tok - mu) ** 2, axis=-1, keepdims=True)
    tok = (tok - mu) / jnp.sqrt(var + 1e-5) * gf_ref[...] + bef_ref[...]
    out_ref[...] = jax.nn.gelu(tok)


def _fuse_call(aggsum, denp3, Wp1, bp1, Wp2, bp2, Wf, bf, gf, bef):
    return pl.pallas_call(
        _fuse_body,
        grid=(MB,),
        in_specs=[
            pl.BlockSpec((BM, D), lambda i: (i, 0)),
            pl.BlockSpec((1, 1, BM), lambda i: (i, 0, 0)),
            pl.BlockSpec((3, P), lambda i: (0, 0)),
            pl.BlockSpec((1, P), lambda i: (0, 0)),
            pl.BlockSpec((P, P), lambda i: (0, 0)),
            pl.BlockSpec((1, P), lambda i: (0, 0)),
            pl.BlockSpec((D + P, D), lambda i: (0, 0)),
            pl.BlockSpec((1, D), lambda i: (0, 0)),
            pl.BlockSpec((1, D), lambda i: (0, 0)),
            pl.BlockSpec((1, D), lambda i: (0, 0)),
        ],
        out_specs=pl.BlockSpec((BM, D), lambda i: (i, 0)),
        out_shape=jax.ShapeDtypeStruct((M, D), jnp.float32),
    )(aggsum, denp3, Wp1, bp1, Wp2, bp2, Wf, bf, gf, bef)


def kernel(feats, points3d, conf_logits, W1, b1, g1, be1, W2, b2,
           Wp1, bp1, Wp2, bp2, Wf, bf, gf, bef):
    conf2 = conf_logits.reshape(N, 1)
    r = lambda v: v.reshape(1, D)
    hw, w2, vid2 = _mlp_call(feats, points3d, conf2, W1, r(b1), r(g1), r(be1),
                             W2, r(b2))
    aggsum, denp = _sc_call(hw, w2.reshape(N), vid2.reshape(N))
    return _fuse_call(aggsum, denp.reshape(M // BM, 1, BM),
                      Wp1, bp1.reshape(1, P), Wp2, bp2.reshape(1, P),
                      Wf, bf.reshape(1, D), gf.reshape(1, D), bef.reshape(1, D))


# speedup vs baseline: 1.7731x; 1.2078x over previous
import functools

import jax
import jax.numpy as jnp
from jax import lax
from jax.experimental import pallas as pl
from jax.experimental.pallas import tpu as pltpu
from jax.experimental.pallas import tpu_sc as plsc

G = 32
M = G * G * G
VS = 1.0 / G
N = 65536
D = 1024
P = 128


BN = 256
NB = N // BN


def _mlp_body(feats_ref, pts_ref, conf_ref, W1_ref, b1_ref, g1_ref, be1_ref,
              W2_ref, b2_ref, hw_ref, w_ref, vid_ref):
    x = feats_ref[...].astype(jnp.bfloat16)
    h = jnp.dot(x, W1_ref[...], preferred_element_type=jnp.float32) + b1_ref[...]
    mu = jnp.mean(h, axis=-1, keepdims=True)
    var = jnp.mean((h - mu) ** 2, axis=-1, keepdims=True)
    h = (h - mu) / jnp.sqrt(var + 1e-5) * g1_ref[...] + be1_ref[...]
    h = jax.nn.gelu(h)
    h = jnp.dot(h.astype(jnp.bfloat16), W2_ref[...],
                preferred_element_type=jnp.float32) + b2_ref[...]
    w = 1.0 + jnp.exp(conf_ref[...])
    hw_ref[...] = h * w
    w_ref[...] = w
    pts = pts_ref[...]
    ij = jnp.clip(jnp.floor(pts / VS), 0, G - 1).astype(jnp.int32)
    vid_ref[...] = ij[:, 0:1] * (G * G) + ij[:, 1:2] * G + ij[:, 2:3]


def _mlp_call(feats, pts, conf2, W1, b1, g1, be1, W2, b2):
    return pl.pallas_call(
        _mlp_body,
        grid=(NB,),
        in_specs=[
            pl.BlockSpec((BN, D), lambda i: (i, 0)),
            pl.BlockSpec((BN, 3), lambda i: (i, 0)),
            pl.BlockSpec((BN, 1), lambda i: (i, 0)),
            pl.BlockSpec((D, D), lambda i: (0, 0)),
            pl.BlockSpec((1, D), lambda i: (0, 0)),
            pl.BlockSpec((1, D), lambda i: (0, 0)),
            pl.BlockSpec((1, D), lambda i: (0, 0)),
            pl.BlockSpec((D, D), lambda i: (0, 0)),
            pl.BlockSpec((1, D), lambda i: (0, 0)),
        ],
        out_specs=[
            pl.BlockSpec((BN, D), lambda i: (i, 0)),
            pl.BlockSpec((BN, 1), lambda i: (i, 0)),
            pl.BlockSpec((BN, 1), lambda i: (i, 0)),
        ],
        out_shape=[
            jax.ShapeDtypeStruct((N, D), jnp.float32),
            jax.ShapeDtypeStruct((N, 1), jnp.float32),
            jax.ShapeDtypeStruct((N, 1), jnp.int32),
        ],
    )(feats, pts, conf2, W1, b1, g1, be1, W2, b2)


NSC = 2
NT = 16
HALF = M // NSC
PPTS = N // NT
BK = 64
NBK = HALF // BK
ROUNDS = NBK // NT
CAPB = 48
LISTW = NBK * CAPB
CTOT = NT * CAPB
CH = 8


def _take16(x, idx):
    dn = lax.GatherDimensionNumbers(offset_dims=(), collapsed_slice_dims=(0,),
                                    start_index_map=(0,))
    return lax.gather(x, idx[:, None], dn, (1,),
                      mode=lax.GatherScatterMode.PROMISE_IN_BOUNDS)


def _sc_body(hw_hbm, w_hbm, vid_hbm, agg_hbm, den_hbm,
             lists_sh, cnts_sh, vid_l, listb, cnts, cbuf, pstage, cpacked,
             pbufg, acc, rowb0, rowb1, wbufall, den64, lsem, wsem, gsem):
    c = lax.axis_index("c")
    s = lax.axis_index("s")
    half0 = c * HALF
    zero16f = jnp.zeros((16,), jnp.float32)
    zero16i = jnp.zeros((16,), jnp.int32)
    iota16 = lax.iota(jnp.int32, 16)

    pltpu.sync_copy(vid_hbm.at[pl.ds(s * PPTS, PPTS)], vid_l)

    def zc(i, _):
        cnts[pl.ds(i * 16, 16)] = zero16i
        return 0
    lax.fori_loop(0, (NBK + 16) // 16, zc, 0)

    def prod(i, _):
        v = vid_l[pl.ds(i * 16, 16)]
        rel = v - half0
        valid = (rel >= 0) & (rel < HALF)
        b = jnp.where(valid, rel >> 6, NBK)
        packed = ((v & (BK - 1)) << 16) | (s * PPTS + i * 16 + iota16)
        sb, sp = plsc.sort_key_val(b, packed)
        prev = _take16(sb, jnp.maximum(iota16 - 1, 0))
        newrun = (sb != prev) | (iota16 == 0)
        runpos = plsc.cummax(jnp.where(newrun, iota16, 0))
        rank = iota16 - runpos
        nxt = _take16(sb, jnp.minimum(iota16 + 1, 15))
        runend = (sb != nxt) | (iota16 == 15)
        cb = plsc.load_gather(cnts, [sb])
        pos = sb * CAPB + jnp.minimum(cb + rank, CAPB - 1)
        plsc.store_scatter(listb, [pos], sp)
        plsc.store_scatter(cnts, [sb], cb + rank + 1, mask=runend)
        return 0
    lax.fori_loop(0, PPTS // 16, prod, 0)

    pltpu.sync_copy(listb.at[pl.ds(0, LISTW)], lists_sh.at[pl.ds(s * LISTW, LISTW)])
    pltpu.sync_copy(cnts.at[pl.ds(0, NBK)], cnts_sh.at[pl.ds(s * NBK, NBK)])
    plsc.subcore_barrier()
    pltpu.sync_copy(cnts_sh, cbuf)

    def rnd(r, _):
        b = r * NT + s
        v0 = half0 + b * BK

        def za(i, _):
            for q in range(D // 16):
                acc[i, pl.ds(q * 16, 16)] = zero16f
            return 0
        lax.fori_loop(0, BK, za, 0)
        for i in range(BK // 16):
            den64[pl.ds(i * 16, 16)] = zero16f

        descs = [pltpu.async_copy(
            lists_sh.at[pl.ds(s2 * LISTW + b * CAPB, CAPB)],
            pstage.at[pl.ds(s2 * CAPB, CAPB)], lsem) for s2 in range(NT)]
        for dsc in descs:
            dsc.wait()
        cvec = plsc.load_gather(cbuf, [iota16 * NBK + b])

        cnt = zero16i
        for k in range(CTOT // 16):
            t = k % (CAPB // 16)
            c2 = jnp.minimum(cvec[k // (CAPB // 16)], CAPB)
            vec = pstage[pl.ds(k * 16, 16)]
            m = (t * 16 + iota16) < c2
            pos = cnt + plsc.cumsum(m.astype(jnp.int32)) - 1
            plsc.store_scatter(cpacked, [pos], vec, mask=m)
            cnt = cnt + plsc.all_reduce_population_count(m)
        ctot = jnp.max(cnt)

        def bg(k, _):
            pbufg[pl.ds(k * 16, 16)] = cpacked[pl.ds(k * 16, 16)] & 0xFFFF
            return 0
        lax.fori_loop(0, CTOT // 16, bg, 0)

        for u in range(CTOT // 128):
            pltpu.async_copy(w_hbm.at[pbufg.at[pl.ds(u * 128, 128)]],
                             wbufall.at[pl.ds(u * 128, 128)], wsem)
        for u in range(CTOT // 128):
            pltpu.make_async_copy(w_hbm.at[pbufg.at[pl.ds(u * 128, 128)]],
                                  wbufall.at[pl.ds(u * 128, 128)], wsem).wait()

        nc = (ctot + CH - 1) // CH

        @pl.when(nc > 0)
        def _prime():
            pltpu.async_copy(hw_hbm.at[pbufg.at[pl.ds(0, CH)]], rowb0, gsem)

        def chunk2(jj, _):
            pkv = cpacked[pl.ds(jj * 16, 16)]
            wv = wbufall[pl.ds(jj * 16, 16)]
            kmax = jnp.minimum(ctot - jj * 16, 16)

            @pl.when(kmax > 0)
            def _den():
                key = jnp.where(iota16 < kmax, pkv >> 16, BK)
                sk, sw = plsc.sort_key_val(key, wv)
                prev = _take16(sk, jnp.maximum(iota16 - 1, 0))
                newrun = (sk != prev) | (iota16 == 0)
                runpos = plsc.cummax(jnp.where(newrun, iota16, 0))
                nxt = _take16(sk, jnp.minimum(iota16 + 1, 15))
                runend = (sk != nxt) | (iota16 == 15)
                cs = plsc.cumsum(sw)
                prevcs = _take16(cs, jnp.maximum(runpos - 1, 0))
                seg = cs - jnp.where(runpos == 0, 0.0, prevcs)
                plsc.addupdate_scatter(den64, [sk], seg,
                                       mask=runend & (sk < BK))

            for par, cur, nxt_b in ((0, rowb0, rowb1), (1, rowb1, rowb0)):
                j = jj * 2 + par

                @pl.when(j < nc)
                def _proc():
                    pltpu.make_async_copy(
                        hw_hbm.at[pbufg.at[pl.ds(j * CH, CH)]], cur,
                        gsem).wait()

                    @pl.when(j + 1 < nc)
                    def _fire():
                        pltpu.async_copy(
                            hw_hbm.at[pbufg.at[pl.ds((j + 1) * CH, CH)]],
                            nxt_b, gsem)

                    for k in range(CH):
                        @pl.when(j * CH + k < ctot)
                        def _add():
                            lv = pkv[par * CH + k] >> 16
                            for q in range(D // 16):
                                plsc.addupdate(acc.at[lv, pl.ds(q * 16, 16)],
                                               cur[k, pl.ds(q * 16, 16)])
            return 0
        lax.fori_loop(0, (nc + 1) // 2, chunk2, 0)

        pltpu.sync_copy(acc, agg_hbm.at[pl.ds(v0, BK)])
        pltpu.sync_copy(den64, den_hbm.at[pl.ds(v0, BK)])
        return 0
    lax.fori_loop(0, ROUNDS, rnd, 0)


def _sc_call(hw, w, vid):
    mesh = plsc.VectorSubcoreMesh(core_axis_name="c", subcore_axis_name="s")
    f = pl.kernel(
        _sc_body,
        out_type=(
            jax.ShapeDtypeStruct((M, D), jnp.float32),
            jax.ShapeDtypeStruct((M,), jnp.float32),
        ),
        mesh=mesh,
        compiler_params=pltpu.CompilerParams(needs_layout_passes=False),
        scratch_types=[
            pltpu.VMEM_SHARED((NT * LISTW,), jnp.int32),
            pltpu.VMEM_SHARED((NT * NBK,), jnp.int32),
            pltpu.VMEM((PPTS,), jnp.int32),
            pltpu.VMEM((LISTW + CAPB,), jnp.int32),
            pltpu.VMEM((NBK + 16,), jnp.int32),
            pltpu.VMEM((NT * NBK,), jnp.int32),
            pltpu.VMEM((CTOT,), jnp.int32),
            pltpu.VMEM((CTOT,), jnp.int32),
            pltpu.VMEM((CTOT,), jnp.int32),
            pltpu.VMEM((BK, D), jnp.float32),
            pltpu.VMEM((CH, D), jnp.float32),
            pltpu.VMEM((CH, D), jnp.float32),
            pltpu.VMEM((CTOT,), jnp.float32),
            pltpu.VMEM((BK,), jnp.float32),
            pltpu.SemaphoreType.DMA,
            pltpu.SemaphoreType.DMA,
            pltpu.SemaphoreType.DMA,
        ],
    )
    return f(hw, w, vid)


BM = 512
MB = M // BM


def _fuse_body(agg_ref, denp_ref, Wp1_ref, bp1_ref, Wp2_ref, bp2_ref,
               Wf_ref, bf_ref, gf_ref, bef_ref, out_ref):
    i = pl.program_id(0)
    den = denp_ref[0, 0]
    agg = agg_ref[...] / (den[:, None] + 1e-8)
    gi = i * BM + lax.broadcasted_iota(jnp.int32, (BM, 1), 0)
    c0 = ((gi // (G * G)).astype(jnp.float32) + 0.5) * VS
    c1 = (((gi // G) % G).astype(jnp.float32) + 0.5) * VS
    c2 = ((gi % G).astype(jnp.float32) + 0.5) * VS
    Wp1 = Wp1_ref[...]
    pe = c0 * Wp1[0:1, :] + c1 * Wp1[1:2, :] + c2 * Wp1[2:3, :] + bp1_ref[...]
    pe = jax.nn.silu(pe)
    pe = jnp.dot(pe, Wp2_ref[...], preferred_element_type=jnp.float32) + bp2_ref[...]
    Wf = Wf_ref[...]
    tok = (jnp.dot(agg.astype(jnp.bfloat16), Wf[:D, :],
                   preferred_element_type=jnp.float32)
           + jnp.dot(pe.astype(jnp.bfloat16), Wf[D:, :],
                     preferred_element_type=jnp.float32)
           + bf_ref[...])
    mu = jnp.mean(tok, axis=-1, keepdims=True)
    var = jnp.mean((tok - mu) ** 2, axis=-1, keepdims=True)
    tok = (tok - mu) / jnp.sqrt(var + 1e-5) * gf_ref[...] + bef_ref[...]
    out_ref[...] = jax.nn.gelu(tok)


def _fuse_call(aggsum, denp3, Wp1, bp1, Wp2, bp2, Wf, bf, gf, bef):
    return pl.pallas_call(
        _fuse_body,
        grid=(MB,),
        in_specs=[
            pl.BlockSpec((BM, D), lambda i: (i, 0)),
            pl.BlockSpec((1, 1, BM), lambda i: (i, 0, 0)),
            pl.BlockSpec((3, P), lambda i: (0, 0)),
            pl.BlockSpec((1, P), lambda i: (0, 0)),
            pl.BlockSpec((P, P), lambda i: (0, 0)),
            pl.BlockSpec((1, P), lambda i: (0, 0)),
            pl.BlockSpec((D + P, D), lambda i: (0, 0)),
            pl.BlockSpec((1, D), lambda i: (0, 0)),
            pl.BlockSpec((1, D), lambda i: (0, 0)),
            pl.BlockSpec((1, D), lambda i: (0, 0)),
        ],
        out_specs=pl.BlockSpec((BM, D), lambda i: (i, 0)),
        out_shape=jax.ShapeDtypeStruct((M, D), jnp.float32),
    )(aggsum, denp3, Wp1, bp1, Wp2, bp2, Wf, bf, gf, bef)


def kernel(feats, points3d, conf_logits, W1, b1, g1, be1, W2, b2,
           Wp1, bp1, Wp2, bp2, Wf, bf, gf, bef):
    conf2 = conf_logits.reshape(N, 1)
    r = lambda v: v.reshape(1, D)
    hw, w2, vid2 = _mlp_call(feats, points3d, conf2, W1.astype(jnp.bfloat16),
                             r(b1), r(g1), r(be1), W2.astype(jnp.bfloat16),
                             r(b2))
    aggsum, denp = _sc_call(hw, w2.reshape(N), vid2.reshape(N))
    return _fuse_call(aggsum, denp.reshape(M // BM, 1, BM),
                      Wp1, bp1.reshape(1, P), Wp2, bp2.reshape(1, P),
                      Wf.astype(jnp.bfloat16), bf.reshape(1, D),
                      gf.reshape(1, D), bef.reshape(1, D))


# speedup vs baseline: 2.0478x; 1.1550x over previous
import functools

import jax
import jax.numpy as jnp
from jax import lax
from jax.experimental import pallas as pl
from jax.experimental.pallas import tpu as pltpu
from jax.experimental.pallas import tpu_sc as plsc

G = 32
M = G * G * G
VS = 1.0 / G
N = 65536
D = 1024
P = 128


BN = 256
NB = N // BN


def _mlp_body(feats_ref, pts_ref, conf_ref, W1_ref, b1_ref, g1_ref, be1_ref,
              W2_ref, b2_ref, hw_ref, w_ref, vid_ref):
    x = feats_ref[...].astype(jnp.bfloat16)
    h = jnp.dot(x, W1_ref[...], preferred_element_type=jnp.float32) + b1_ref[...]
    mu = jnp.mean(h, axis=-1, keepdims=True)
    var = jnp.mean((h - mu) ** 2, axis=-1, keepdims=True)
    h = (h - mu) / jnp.sqrt(var + 1e-5) * g1_ref[...] + be1_ref[...]
    h = jax.nn.gelu(h)
    h = jnp.dot(h.astype(jnp.bfloat16), W2_ref[...],
                preferred_element_type=jnp.float32) + b2_ref[...]
    w = 1.0 + jnp.exp(conf_ref[...])
    hw_ref[...] = h * w
    w_ref[...] = w
    pts = pts_ref[...]
    ij = jnp.clip(jnp.floor(pts / VS), 0, G - 1).astype(jnp.int32)
    vid_ref[...] = ij[:, 0:1] * (G * G) + ij[:, 1:2] * G + ij[:, 2:3]


def _mlp_call(feats, pts, conf2, W1, b1, g1, be1, W2, b2):
    return pl.pallas_call(
        _mlp_body,
        grid=(NB,),
        in_specs=[
            pl.BlockSpec((BN, D), lambda i: (i, 0)),
            pl.BlockSpec((BN, 3), lambda i: (i, 0)),
            pl.BlockSpec((BN, 1), lambda i: (i, 0)),
            pl.BlockSpec((D, D), lambda i: (0, 0)),
            pl.BlockSpec((1, D), lambda i: (0, 0)),
            pl.BlockSpec((1, D), lambda i: (0, 0)),
            pl.BlockSpec((1, D), lambda i: (0, 0)),
            pl.BlockSpec((D, D), lambda i: (0, 0)),
            pl.BlockSpec((1, D), lambda i: (0, 0)),
        ],
        out_specs=[
            pl.BlockSpec((BN, D), lambda i: (i, 0)),
            pl.BlockSpec((BN, 1), lambda i: (i, 0)),
            pl.BlockSpec((BN, 1), lambda i: (i, 0)),
        ],
        out_shape=[
            jax.ShapeDtypeStruct((N, D), jnp.float32),
            jax.ShapeDtypeStruct((N, 1), jnp.float32),
            jax.ShapeDtypeStruct((N, 1), jnp.int32),
        ],
    )(feats, pts, conf2, W1, b1, g1, be1, W2, b2)


NSC = 2
NT = 16
HALF = M // NSC
PPTS = N // NT
BK = 64
NBK = HALF // BK
ROUNDS = NBK // NT
CAPB = 48
LISTW = NBK * CAPB
CTOT = NT * CAPB
CH = 8


def _take16(x, idx):
    dn = lax.GatherDimensionNumbers(offset_dims=(), collapsed_slice_dims=(0,),
                                    start_index_map=(0,))
    return lax.gather(x, idx[:, None], dn, (1,),
                      mode=lax.GatherScatterMode.PROMISE_IN_BOUNDS)


def _sc_body(hw_hbm, w_hbm, vid_hbm, agg_hbm, den_hbm,
             lists_sh, cnts_sh, vid_l, listb, cnts, cbuf, pstage, cpacked,
             pbufg, acc, rowb0, rowb1, wbufall, den64, lsem, wsem, gsem):
    c = lax.axis_index("c")
    s = lax.axis_index("s")
    half0 = c * HALF
    zero16f = jnp.zeros((16,), jnp.float32)
    zero16i = jnp.zeros((16,), jnp.int32)
    iota16 = lax.iota(jnp.int32, 16)

    pltpu.sync_copy(vid_hbm.at[pl.ds(s * PPTS, PPTS)], vid_l)

    def zc(i, _):
        cnts[pl.ds(i * 16, 16)] = zero16i
        return 0
    lax.fori_loop(0, (NBK + 16) // 16, zc, 0)

    def prod(i, _):
        v = vid_l[pl.ds(i * 16, 16)]
        rel = v - half0
        valid = (rel >= 0) & (rel < HALF)
        b = jnp.where(valid, rel >> 6, NBK)
        packed = ((v & (BK - 1)) << 16) | (s * PPTS + i * 16 + iota16)
        sb, sp = plsc.sort_key_val(b, packed)
        prev = _take16(sb, jnp.maximum(iota16 - 1, 0))
        newrun = (sb != prev) | (iota16 == 0)
        runpos = plsc.cummax(jnp.where(newrun, iota16, 0))
        rank = iota16 - runpos
        nxt = _take16(sb, jnp.minimum(iota16 + 1, 15))
        runend = (sb != nxt) | (iota16 == 15)
        cb = plsc.load_gather(cnts, [sb])
        pos = sb * CAPB + jnp.minimum(cb + rank, CAPB - 1)
        plsc.store_scatter(listb, [pos], sp)
        plsc.store_scatter(cnts, [sb], cb + rank + 1, mask=runend)
        return 0
    lax.fori_loop(0, PPTS // 16, prod, 0)

    pltpu.sync_copy(listb.at[pl.ds(0, LISTW)], lists_sh.at[pl.ds(s * LISTW, LISTW)])
    pltpu.sync_copy(cnts.at[pl.ds(0, NBK)], cnts_sh.at[pl.ds(s * NBK, NBK)])
    plsc.subcore_barrier()
    pltpu.sync_copy(cnts_sh, cbuf)

    def rnd(r, _):
        b = r * NT + s
        v0 = half0 + b * BK

        def za(i, _):
            for q in range(D // 16):
                acc[i, pl.ds(q * 16, 16)] = zero16f
            return 0
        lax.fori_loop(0, BK, za, 0)
        for i in range(BK // 16):
            den64[pl.ds(i * 16, 16)] = zero16f

        descs = [pltpu.async_copy(
            lists_sh.at[pl.ds(s2 * LISTW + b * CAPB, CAPB)],
            pstage.at[pl.ds(s2 * CAPB, CAPB)], lsem) for s2 in range(NT)]
        for dsc in descs:
            dsc.wait()
        cvec = plsc.load_gather(cbuf, [iota16 * NBK + b])

        cnt = zero16i
        for k in range(CTOT // 16):
            t = k % (CAPB // 16)
            c2 = jnp.minimum(cvec[k // (CAPB // 16)], CAPB)
            vec = pstage[pl.ds(k * 16, 16)]
            m = (t * 16 + iota16) < c2
            pos = cnt + plsc.cumsum(m.astype(jnp.int32)) - 1
            plsc.store_scatter(cpacked, [pos], vec, mask=m)
            cnt = cnt + plsc.all_reduce_population_count(m)
        ctot = jnp.max(cnt)

        def bg(k, _):
            pbufg[pl.ds(k * 16, 16)] = cpacked[pl.ds(k * 16, 16)] & 0xFFFF
            return 0
        lax.fori_loop(0, CTOT // 16, bg, 0)

        nc = (ctot + CH - 1) // CH

        padpos = ctot + iota16
        plsc.store_scatter(cpacked, [padpos],
                           jnp.full((16,), BK << 16, jnp.int32),
                           mask=padpos < nc * CH)

        def bg2(k, _):
            pbufg[pl.ds(k * 16, 16)] = cpacked[pl.ds(k * 16, 16)] & 0xFFFF
            return 0
        lax.fori_loop(0, (nc * CH + 15) // 16, bg2, 0)

        for u in range(CTOT // 128):
            pltpu.async_copy(w_hbm.at[pbufg.at[pl.ds(u * 128, 128)]],
                             wbufall.at[pl.ds(u * 128, 128)], wsem)

        @pl.when(nc > 0)
        def _prime():
            pltpu.async_copy(hw_hbm.at[pbufg.at[pl.ds(0, CH)]], rowb0, gsem)

        def chunk2(jj, _):
            pkv = cpacked[pl.ds(jj * 16, 16)]

            for par, cur, nxt_b in ((0, rowb0, rowb1), (1, rowb1, rowb0)):
                j = jj * 2 + par

                @pl.when(j < nc)
                def _proc():
                    pltpu.make_async_copy(
                        hw_hbm.at[pbufg.at[pl.ds(j * CH, CH)]], cur,
                        gsem).wait()

                    @pl.when(j + 1 < nc)
                    def _fire():
                        pltpu.async_copy(
                            hw_hbm.at[pbufg.at[pl.ds((j + 1) * CH, CH)]],
                            nxt_b, gsem)

                    for k in range(CH):
                        lv = pkv[par * CH + k] >> 16
                        for qb in range(D // 256):
                            vals = [cur[k, pl.ds((qb * 16 + q) * 16, 16)]
                                    for q in range(16)]
                            for q in range(16):
                                plsc.addupdate(
                                    acc.at[lv, pl.ds((qb * 16 + q) * 16, 16)],
                                    vals[q])
            return 0
        lax.fori_loop(0, (nc + 1) // 2, chunk2, 0)

        for u in range(CTOT // 128):
            pltpu.make_async_copy(w_hbm.at[pbufg.at[pl.ds(u * 128, 128)]],
                                  wbufall.at[pl.ds(u * 128, 128)], wsem).wait()

        def denstep(jj, _):
            pkv = cpacked[pl.ds(jj * 16, 16)]
            wv = wbufall[pl.ds(jj * 16, 16)]
            kmax = jnp.minimum(ctot - jj * 16, 16)
            key = jnp.where(iota16 < kmax, pkv >> 16, BK)
            sk, sw = plsc.sort_key_val(key, wv)
            prev = _take16(sk, jnp.maximum(iota16 - 1, 0))
            newrun = (sk != prev) | (iota16 == 0)
            runpos = plsc.cummax(jnp.where(newrun, iota16, 0))
            nxt = _take16(sk, jnp.minimum(iota16 + 1, 15))
            runend = (sk != nxt) | (iota16 == 15)
            cs = plsc.cumsum(sw)
            prevcs = _take16(cs, jnp.maximum(runpos - 1, 0))
            seg = cs - jnp.where(runpos == 0, 0.0, prevcs)
            plsc.addupdate_scatter(den64, [sk], seg,
                                   mask=runend & (sk < BK))
            return 0
        lax.fori_loop(0, (ctot + 15) // 16, denstep, 0)

        pltpu.sync_copy(acc.at[pl.ds(0, BK)], agg_hbm.at[pl.ds(v0, BK)])
        pltpu.sync_copy(den64, den_hbm.at[pl.ds(v0, BK)])
        return 0
    lax.fori_loop(0, ROUNDS, rnd, 0)


def _sc_call(hw, w, vid):
    mesh = plsc.VectorSubcoreMesh(core_axis_name="c", subcore_axis_name="s")
    f = pl.kernel(
        _sc_body,
        out_type=(
            jax.ShapeDtypeStruct((M, D), jnp.float32),
            jax.ShapeDtypeStruct((M,), jnp.float32),
        ),
        mesh=mesh,
        compiler_params=pltpu.CompilerParams(needs_layout_passes=False),
        scratch_types=[
            pltpu.VMEM_SHARED((NT * LISTW,), jnp.int32),
            pltpu.VMEM_SHARED((NT * NBK,), jnp.int32),
            pltpu.VMEM((PPTS,), jnp.int32),
            pltpu.VMEM((LISTW + CAPB,), jnp.int32),
            pltpu.VMEM((NBK + 16,), jnp.int32),
            pltpu.VMEM((NT * NBK,), jnp.int32),
            pltpu.VMEM((CTOT,), jnp.int32),
            pltpu.VMEM((CTOT + 16,), jnp.int32),
            pltpu.VMEM((CTOT + 16,), jnp.int32),
            pltpu.VMEM((BK + 1, D), jnp.float32),
            pltpu.VMEM((CH, D), jnp.float32),
            pltpu.VMEM((CH, D), jnp.float32),
            pltpu.VMEM((CTOT,), jnp.float32),
            pltpu.VMEM((BK,), jnp.float32),
            pltpu.SemaphoreType.DMA,
            pltpu.SemaphoreType.DMA,
            pltpu.SemaphoreType.DMA,
        ],
    )
    return f(hw, w, vid)


BM = 512
MB = M // BM


def _fuse_body(agg_ref, denp_ref, Wp1_ref, bp1_ref, Wp2_ref, bp2_ref,
               Wf_ref, bf_ref, gf_ref, bef_ref, out_ref):
    i = pl.program_id(0)
    den = denp_ref[0, 0]
    agg = agg_ref[...] / (den[:, None] + 1e-8)
    gi = i * BM + lax.broadcasted_iota(jnp.int32, (BM, 1), 0)
    c0 = ((gi // (G * G)).astype(jnp.float32) + 0.5) * VS
    c1 = (((gi // G) % G).astype(jnp.float32) + 0.5) * VS
    c2 = ((gi % G).astype(jnp.float32) + 0.5) * VS
    Wp1 = Wp1_ref[...]
    pe = c0 * Wp1[0:1, :] + c1 * Wp1[1:2, :] + c2 * Wp1[2:3, :] + bp1_ref[...]
    pe = jax.nn.silu(pe)
    pe = jnp.dot(pe, Wp2_ref[...], preferred_element_type=jnp.float32) + bp2_ref[...]
    Wf = Wf_ref[...]
    tok = (jnp.dot(agg.astype(jnp.bfloat16), Wf[:D, :],
                   preferred_element_type=jnp.float32)
           + jnp.dot(pe.astype(jnp.bfloat16), Wf[D:, :],
                     preferred_element_type=jnp.float32)
           + bf_ref[...])
    mu = jnp.mean(tok, axis=-1, keepdims=True)
    var = jnp.mean((tok - mu) ** 2, axis=-1, keepdims=True)
    tok = (tok - mu) / jnp.sqrt(var + 1e-5) * gf_ref[...] + bef_ref[...]
    out_ref[...] = jax.nn.gelu(tok)


def _fuse_call(aggsum, denp3, Wp1, bp1, Wp2, bp2, Wf, bf, gf, bef):
    return pl.pallas_call(
        _fuse_body,
        grid=(MB,),
        in_specs=[
            pl.BlockSpec((BM, D), lambda i: (i, 0)),
            pl.BlockSpec((1, 1, BM), lambda i: (i, 0, 0)),
            pl.BlockSpec((3, P), lambda i: (0, 0)),
            pl.BlockSpec((1, P), lambda i: (0, 0)),
            pl.BlockSpec((P, P), lambda i: (0, 0)),
            pl.BlockSpec((1, P), lambda i: (0, 0)),
            pl.BlockSpec((D + P, D), lambda i: (0, 0)),
            pl.BlockSpec((1, D), lambda i: (0, 0)),
            pl.BlockSpec((1, D), lambda i: (0, 0)),
            pl.BlockSpec((1, D), lambda i: (0, 0)),
        ],
        out_specs=pl.BlockSpec((BM, D), lambda i: (i, 0)),
        out_shape=jax.ShapeDtypeStruct((M, D), jnp.float32),
    )(aggsum, denp3, Wp1, bp1, Wp2, bp2, Wf, bf, gf, bef)


def kernel(feats, points3d, conf_logits, W1, b1, g1, be1, W2, b2,
           Wp1, bp1, Wp2, bp2, Wf, bf, gf, bef):
    conf2 = conf_logits.reshape(N, 1)
    r = lambda v: v.reshape(1, D)
    hw, w2, vid2 = _mlp_call(feats, points3d, conf2, W1.astype(jnp.bfloat16),
                             r(b1), r(g1), r(be1), W2.astype(jnp.bfloat16),
                             r(b2))
    aggsum, denp = _sc_call(hw, w2.reshape(N), vid2.reshape(N))
    return _fuse_call(aggsum, denp.reshape(M // BM, 1, BM),
                      Wp1, bp1.reshape(1, P), Wp2, bp2.reshape(1, P),
                      Wf.astype(jnp.bfloat16), bf.reshape(1, D),
                      gf.reshape(1, D), bef.reshape(1, D))
